# Initial kernel scaffold; baseline (speedup 1.0000x reference)
import jax
import jax.numpy as jnp
from jax.experimental import pallas as pl


def kernel(logits, targets):
    raise NotImplementedError("write your pallas kernel here")


# speedup vs baseline: 31.1771x; 31.1771x over previous
import functools

import numpy as np
import jax
import jax.numpy as jnp
from jax import lax
from jax.experimental import pallas as pl
from jax.experimental.pallas import tpu as pltpu
from jax.experimental.pallas import tpu_sc as plsc

_B, _C, _H, _W = 8, 19, 512, 512
_P = _H * _W
_N = _B * _P
_L = _B * _C * _P
_HB = 32
_NHB = _H // _HB
_SMOOTH = 1e-06

_NW = 32
_RW = _N // _NW
_ROUND = 1024
_NR = _RW // _ROUND
_NBINS = _C * 16


def _tc_body(x_ref, probs_ref, g_ref):
    b = pl.program_id(1)
    x = x_ref[0]
    m = jnp.max(x, axis=0)
    e = jnp.exp(x - m[None])
    s = jnp.sum(e, axis=0)
    p = e * (1.0 / s)[None]
    probs_ref[0] = p

    row_a = lax.broadcasted_iota(jnp.int32, (_C, _C * _HB), 0)
    col = lax.broadcasted_iota(jnp.int32, (_C, _C * _HB), 1)
    m0 = (((col // _HB) - (col % _HB)) % _C == row_a).astype(jnp.float32)
    p2 = p.reshape(_C * _HB, _W)
    a0 = lax.dot_general(m0, p2, (((1,), (0,)), ((), ())),
                         preferred_element_type=jnp.float32)
    wrow = lax.broadcasted_iota(jnp.int32, (_W, _C), 0)
    wcol = lax.broadcasted_iota(jnp.int32, (_W, _C), 1)
    w2 = (wrow % _C == wcol).astype(jnp.float32)
    g = lax.dot_general(a0, w2, (((1,), (0,)), ((), ())),
                        preferred_element_type=jnp.float32)

    @pl.when(b == 0)
    def _():
        g_ref[0] = jnp.zeros((_C, _C), jnp.float32)

    g_ref[0] += g


_tc_call = pl.pallas_call(
    _tc_body,
    grid=(_NHB, _B),
    in_specs=[pl.BlockSpec((1, _C, _HB, _W), lambda hb, b: (b, 0, hb, 0))],
    out_specs=[pl.BlockSpec((1, _C, _HB, _W), lambda hb, b: (b, 0, hb, 0)),
               pl.BlockSpec((1, _C, _C), lambda hb, b: (hb, 0, 0))],
    out_shape=[jax.ShapeDtypeStruct((_B, _C, _H, _W), jnp.float32),
               jax.ShapeDtypeStruct((_NHB, _C, _C), jnp.float32)],
    compiler_params=pltpu.CompilerParams(
        dimension_semantics=("arbitrary", "arbitrary")),
)

_SEL = np.zeros((_NHB, _C, _C, _C), np.float32)
for _hb in range(_NHB):
    for _a in range(_C):
        for _d in range(_C):
            _SEL[_hb, _a, _d, (_a + _d - 32 * _hb) % _C] = 1.0


def _sc_body(probs_hbm, t_hbm, iacc_hbm, cacc_hbm,
             t_v, idx_v, val_v, acc_v, cnt_v, tsem, gsem):
    wid = lax.axis_index("s") * 2 + lax.axis_index("c")
    base = wid * _RW

    for u in range(_NBINS // 16):
        acc_v[pl.ds(u * 16, 16)] = jnp.zeros((16,), jnp.float32)
        cnt_v[pl.ds(u * 16, 16)] = jnp.zeros((16,), jnp.float32)

    lane = lax.iota(jnp.int32, 16)
    ones = jnp.ones((16,), jnp.float32)

    def round_body(r, carry):
        start = base + r * _ROUND
        pltpu.async_copy(t_hbm.at[pl.ds(start, _ROUND)], t_v, tsem).wait()
        for v in range(_ROUND // 16):
            tv = t_v[pl.ds(v * 16, 16)]
            q = start + v * 16 + lane
            idx_v[pl.ds(v * 16, 16)] = q * 19 + tv
        cps = [
            pltpu.async_copy(
                probs_hbm.at[idx_v.at[pl.ds(j * 128, 128)]],
                val_v.at[pl.ds(j * 128, 128)], gsem)
            for j in range(_ROUND // 128)
        ]
        for cp in cps:
            cp.wait()
        for v in range(_ROUND // 16):
            tv = t_v[pl.ds(v * 16, 16)]
            val = val_v[pl.ds(v * 16, 16)]
            sidx = tv * 16 + lane
            plsc.addupdate_scatter(acc_v, [sidx], val)
            plsc.addupdate_scatter(cnt_v, [sidx], ones)
        return carry

    lax.fori_loop(0, _NR, round_body, 0)

    pltpu.sync_copy(acc_v, iacc_hbm.at[wid])
    pltpu.sync_copy(cnt_v, cacc_hbm.at[wid])


@functools.lru_cache(maxsize=None)
def _sc_call_cached():
    return pl.kernel(
        _sc_body,
        out_type=[jax.ShapeDtypeStruct((_NW, _NBINS), jnp.float32),
                  jax.ShapeDtypeStruct((_NW, _NBINS), jnp.float32)],
        mesh=plsc.VectorSubcoreMesh(core_axis_name="c", subcore_axis_name="s",
                                    num_cores=2, num_subcores=16),
        compiler_params=pltpu.CompilerParams(needs_layout_passes=False),
        scratch_types=[
            pltpu.VMEM((_ROUND,), jnp.int32),
            pltpu.VMEM((_ROUND,), jnp.int32),
            pltpu.VMEM((_ROUND,), jnp.float32),
            pltpu.VMEM((_NBINS,), jnp.float32),
            pltpu.VMEM((_NBINS,), jnp.float32),
            pltpu.SemaphoreType.DMA,
            pltpu.SemaphoreType.DMA,
        ],
    )


def kernel(logits, targets):
    probs, gacc = _tc_call(logits)
    probs1d = jnp.reshape(probs, (_L,))
    t1d = jnp.reshape(targets, (_N,))
    iacc, cacc = _sc_call_cached()(probs1d, t1d)
    inter = iacc.reshape(_NW, _C, 16).sum(axis=(0, 2))
    cnt = cacc.reshape(_NW, _C, 16).sum(axis=(0, 2))
    colsum = jnp.einsum("had,hadk->k", gacc, jnp.asarray(_SEL))
    union = colsum + cnt
    dice = 1.0 - (2.0 * inter + _SMOOTH) / (union + _SMOOTH)
    return dice.mean()


# speedup vs baseline: 40.3097x; 1.2929x over previous
import functools

import numpy as np
import jax
import jax.numpy as jnp
from jax import lax
from jax.experimental import pallas as pl
from jax.experimental.pallas import tpu as pltpu
from jax.experimental.pallas import tpu_sc as plsc

_B, _C, _H, _W = 8, 19, 512, 512
_P = _H * _W
_N = _B * _P
_L = _B * _C * _P
_HB = 32
_NHB = _H // _HB
_SMOOTH = 1e-06

_NW = 32
_RW = _N // _NW
_ROUND = 1024
_NR = _RW // _ROUND
_NBINS = _C * 16


def _tc_body(x_ref, probs_ref, g_ref):
    b = pl.program_id(1)
    x = x_ref[0]
    m = jnp.max(x, axis=0)
    e = jnp.exp(x - m[None])
    s = jnp.sum(e, axis=0)
    p = e * (1.0 / s)[None]
    probs_ref[...] = p.reshape(_C, 128, 128)

    row_a = lax.broadcasted_iota(jnp.int32, (_C, _C * _HB), 0)
    col = lax.broadcasted_iota(jnp.int32, (_C, _C * _HB), 1)
    m0 = (((col // _HB) - (col % _HB)) % _C == row_a).astype(jnp.float32)
    p2 = p.reshape(_C * _HB, _W)
    a0 = lax.dot_general(m0, p2, (((1,), (0,)), ((), ())),
                         preferred_element_type=jnp.float32)
    wrow = lax.broadcasted_iota(jnp.int32, (_W, _C), 0)
    wcol = lax.broadcasted_iota(jnp.int32, (_W, _C), 1)
    w2 = (wrow % _C == wcol).astype(jnp.float32)
    g = lax.dot_general(a0, w2, (((1,), (0,)), ((), ())),
                        preferred_element_type=jnp.float32)

    @pl.when(b == 0)
    def _():
        g_ref[0] = jnp.zeros((_C, _C), jnp.float32)

    g_ref[0] += g


_tc_call = pl.pallas_call(
    _tc_body,
    grid=(_NHB, _B),
    in_specs=[pl.BlockSpec((1, _C, _HB, _W), lambda hb, b: (b, 0, hb, 0))],
    out_specs=[pl.BlockSpec((_C, 128, 128), lambda hb, b: (b, hb, 0)),
               pl.BlockSpec((1, _C, _C), lambda hb, b: (hb, 0, 0))],
    out_shape=[jax.ShapeDtypeStruct((_B * _C, _H * _W // 128, 128), jnp.float32),
               jax.ShapeDtypeStruct((_NHB, _C, _C), jnp.float32)],
    compiler_params=pltpu.CompilerParams(
        dimension_semantics=("arbitrary", "arbitrary")),
)

_SEL = np.zeros((_NHB, _C, _C, _C), np.float32)
for _hb in range(_NHB):
    for _a in range(_C):
        for _d in range(_C):
            _SEL[_hb, _a, _d, (_a + _d - 32 * _hb) % _C] = 1.0


def _sc_body(probs_hbm, t_hbm, iacc_hbm, cacc_hbm,
             t_v, idx_v, val_v, acc_v, cnt_v, tsem, gsem):
    wid = lax.axis_index("s") * 2 + lax.axis_index("c")
    base = wid * _RW

    for u in range(_NBINS // 16):
        acc_v[pl.ds(u * 16, 16)] = jnp.zeros((16,), jnp.float32)
        cnt_v[pl.ds(u * 16, 16)] = jnp.zeros((16,), jnp.float32)

    lane = lax.iota(jnp.int32, 16)
    ones = jnp.ones((16,), jnp.float32)

    def round_body(r, carry):
        start = base + r * _ROUND
        pltpu.async_copy(t_hbm.at[pl.ds(start, _ROUND)], t_v, tsem).wait()
        for v in range(_ROUND // 16):
            tv = t_v[pl.ds(v * 16, 16)]
            q = start + v * 16 + lane
            idx_v[pl.ds(v * 16, 16)] = q * 19 + tv
        cps = [
            pltpu.async_copy(
                probs_hbm.at[idx_v.at[pl.ds(j * 128, 128)]],
                val_v.at[pl.ds(j * 128, 128)], gsem)
            for j in range(_ROUND // 128)
        ]
        for cp in cps:
            cp.wait()
        for v in range(_ROUND // 16):
            tv = t_v[pl.ds(v * 16, 16)]
            val = val_v[pl.ds(v * 16, 16)]
            sidx = tv * 16 + lane
            plsc.addupdate_scatter(acc_v, [sidx], val)
            plsc.addupdate_scatter(cnt_v, [sidx], ones)
        return carry

    lax.fori_loop(0, _NR, round_body, 0)

    pltpu.sync_copy(acc_v, iacc_hbm.at[wid])
    pltpu.sync_copy(cnt_v, cacc_hbm.at[wid])


@functools.lru_cache(maxsize=None)
def _sc_call_cached():
    return pl.kernel(
        _sc_body,
        out_type=[jax.ShapeDtypeStruct((_NW, _NBINS), jnp.float32),
                  jax.ShapeDtypeStruct((_NW, _NBINS), jnp.float32)],
        mesh=plsc.VectorSubcoreMesh(core_axis_name="c", subcore_axis_name="s",
                                    num_cores=2, num_subcores=16),
        compiler_params=pltpu.CompilerParams(needs_layout_passes=False),
        scratch_types=[
            pltpu.VMEM((_ROUND,), jnp.int32),
            pltpu.VMEM((_ROUND,), jnp.int32),
            pltpu.VMEM((_ROUND,), jnp.float32),
            pltpu.VMEM((_NBINS,), jnp.float32),
            pltpu.VMEM((_NBINS,), jnp.float32),
            pltpu.SemaphoreType.DMA,
            pltpu.SemaphoreType.DMA,
        ],
    )


def kernel(logits, targets):
    probs3, gacc = _tc_call(logits)
    probs1d = jnp.reshape(probs3, (_L,))
    t1d = jnp.reshape(targets, (_N,))
    iacc, cacc = _sc_call_cached()(probs1d, t1d)
    inter = iacc.reshape(_NW, _C, 16).sum(axis=(0, 2))
    cnt = cacc.reshape(_NW, _C, 16).sum(axis=(0, 2))
    colsum = jnp.einsum("had,hadk->k", gacc, jnp.asarray(_SEL))
    union = colsum + cnt
    dice = 1.0 - (2.0 * inter + _SMOOTH) / (union + _SMOOTH)
    return dice.mean()


# speedup vs baseline: 54.5629x; 1.3536x over previous
import functools

import numpy as np
import jax
import jax.numpy as jnp
from jax import lax
from jax.experimental import pallas as pl
from jax.experimental.pallas import tpu as pltpu
from jax.experimental.pallas import tpu_sc as plsc

_B, _C, _H, _W = 8, 19, 512, 512
_P = _H * _W
_N = _B * _P
_L = _B * _C * _P
_HB = 32
_NHB = _H // _HB
_SMOOTH = 1e-06

_NW = 32
_RW = _N // _NW
_ROUND = 1024
_NR = _RW // _ROUND
_NBINS = _C * 16


def _tc_body(x_ref, probs_ref, g_ref):
    b = pl.program_id(1)
    x = x_ref[0]
    m = jnp.max(x, axis=0)
    e = jnp.exp(x - m[None])
    s = jnp.sum(e, axis=0)
    p = e * (1.0 / s)[None]
    probs_ref[...] = p.reshape(_C, 128, 128)

    row_a = lax.broadcasted_iota(jnp.int32, (_C, _C * _HB), 0)
    col = lax.broadcasted_iota(jnp.int32, (_C, _C * _HB), 1)
    m0 = (((col // _HB) - (col % _HB)) % _C == row_a).astype(jnp.float32)
    p2 = p.reshape(_C * _HB, _W)
    a0 = lax.dot_general(m0, p2, (((1,), (0,)), ((), ())),
                         preferred_element_type=jnp.float32)
    wrow = lax.broadcasted_iota(jnp.int32, (_W, _C), 0)
    wcol = lax.broadcasted_iota(jnp.int32, (_W, _C), 1)
    w2 = (wrow % _C == wcol).astype(jnp.float32)
    g = lax.dot_general(a0, w2, (((1,), (0,)), ((), ())),
                        preferred_element_type=jnp.float32)

    @pl.when(b == 0)
    def _():
        g_ref[0] = jnp.zeros((_C, _C), jnp.float32)

    g_ref[0] += g


_tc_call = pl.pallas_call(
    _tc_body,
    grid=(_NHB, _B),
    in_specs=[pl.BlockSpec((1, _C, _HB, _W), lambda hb, b: (b, 0, hb, 0))],
    out_specs=[pl.BlockSpec((_C, 128, 128), lambda hb, b: (b, hb, 0)),
               pl.BlockSpec((1, _C, _C), lambda hb, b: (hb, 0, 0))],
    out_shape=[jax.ShapeDtypeStruct((_B * _C, _H * _W // 128, 128), jnp.float32),
               jax.ShapeDtypeStruct((_NHB, _C, _C), jnp.float32)],
    compiler_params=pltpu.CompilerParams(
        dimension_semantics=("arbitrary", "arbitrary")),
)

_SEL = np.zeros((_NHB, _C, _C, _C), np.float32)
for _hb in range(_NHB):
    for _a in range(_C):
        for _d in range(_C):
            _SEL[_hb, _a, _d, (_a + _d - 32 * _hb) % _C] = 1.0


def _sc_body(probs_hbm, t_hbm, iacc_hbm, cacc_hbm,
             t_v, idx_v, sidx_v, val_v, acc_v, cnt_v,
             tsem0, tsem1, gsem0, gsem1):
    wid = lax.axis_index("s") * 2 + lax.axis_index("c")
    base = wid * _RW
    tsems = (tsem0, tsem1)
    gsems = (gsem0, gsem1)

    for u in range(_NBINS // 16):
        acc_v[pl.ds(u * 16, 16)] = jnp.zeros((16,), jnp.float32)
        cnt_v[pl.ds(u * 16, 16)] = jnp.zeros((16,), jnp.float32)

    lane = lax.iota(jnp.int32, 16)
    ones = jnp.ones((16,), jnp.float32)

    def tstart(r, slot):
        pltpu.async_copy(t_hbm.at[pl.ds(base + r * _ROUND, _ROUND)],
                         t_v.at[slot], tsems[slot])

    def twait(r, slot):
        pltpu.make_async_copy(t_hbm.at[pl.ds(base + r * _ROUND, _ROUND)],
                              t_v.at[slot], tsems[slot]).wait()

    def compidx(r, slot):
        start = base + r * _ROUND
        for v in range(_ROUND // 16):
            tv = t_v[slot, pl.ds(v * 16, 16)]
            q = start + v * 16 + lane
            idx_v[slot, pl.ds(v * 16, 16)] = q * 19 + tv
            sidx_v[slot, pl.ds(v * 16, 16)] = tv * 16 + lane

    def fire(slot):
        for j in range(_ROUND // 128):
            pltpu.async_copy(
                probs_hbm.at[idx_v.at[slot, pl.ds(j * 128, 128)]],
                val_v.at[slot, pl.ds(j * 128, 128)], gsems[slot])

    def drain(slot):
        for j in range(_ROUND // 128):
            pltpu.make_async_copy(
                probs_hbm.at[idx_v.at[slot, pl.ds(j * 128, 128)]],
                val_v.at[slot, pl.ds(j * 128, 128)], gsems[slot]).wait()

    def process(slot):
        for v in range(_ROUND // 16):
            sidx = sidx_v[slot, pl.ds(v * 16, 16)]
            val = val_v[slot, pl.ds(v * 16, 16)]
            plsc.addupdate_scatter(acc_v, [sidx], val)
            plsc.addupdate_scatter(cnt_v, [sidx], ones)

    def stage_a(r, slot, prefetch_next):
        twait(r, slot)
        compidx(r, slot)
        fire(slot)
        if prefetch_next:
            tstart(r + 1, 1 - slot)

    def stage_b(slot):
        drain(slot)
        process(slot)

    tstart(0, 0)
    stage_a(0, 0, True)

    def pair(pr, carry):
        r1 = 2 * pr + 1
        stage_a(r1, 1, True)
        stage_b(0)
        stage_a(r1 + 1, 0, True)
        stage_b(1)
        return carry

    lax.fori_loop(0, _NR // 2 - 1, pair, 0)

    stage_a(_NR - 1, 1, False)
    stage_b(0)
    stage_b(1)

    pltpu.sync_copy(acc_v, iacc_hbm.at[wid])
    pltpu.sync_copy(cnt_v, cacc_hbm.at[wid])


@functools.lru_cache(maxsize=None)
def _sc_call_cached():
    return pl.kernel(
        _sc_body,
        out_type=[jax.ShapeDtypeStruct((_NW, _NBINS), jnp.float32),
                  jax.ShapeDtypeStruct((_NW, _NBINS), jnp.float32)],
        mesh=plsc.VectorSubcoreMesh(core_axis_name="c", subcore_axis_name="s",
                                    num_cores=2, num_subcores=16),
        compiler_params=pltpu.CompilerParams(needs_layout_passes=False),
        scratch_types=[
            pltpu.VMEM((2, _ROUND), jnp.int32),
            pltpu.VMEM((2, _ROUND), jnp.int32),
            pltpu.VMEM((2, _ROUND), jnp.int32),
            pltpu.VMEM((2, _ROUND), jnp.float32),
            pltpu.VMEM((_NBINS,), jnp.float32),
            pltpu.VMEM((_NBINS,), jnp.float32),
            pltpu.SemaphoreType.DMA,
            pltpu.SemaphoreType.DMA,
            pltpu.SemaphoreType.DMA,
            pltpu.SemaphoreType.DMA,
        ],
    )


def kernel(logits, targets):
    probs3, gacc = _tc_call(logits)
    probs1d = jnp.reshape(probs3, (_L,))
    t1d = jnp.reshape(targets, (_N,))
    iacc, cacc = _sc_call_cached()(probs1d, t1d)
    inter = iacc.reshape(_NW, _C, 16).sum(axis=(0, 2))
    cnt = cacc.reshape(_NW, _C, 16).sum(axis=(0, 2))
    colsum = jnp.einsum("had,hadk->k", gacc, jnp.asarray(_SEL))
    union = colsum + cnt
    dice = 1.0 - (2.0 * inter + _SMOOTH) / (union + _SMOOTH)
    return dice.mean()


# speedup vs baseline: 59.4151x; 1.0889x over previous
import functools

import numpy as np
import jax
import jax.numpy as jnp
from jax import lax
from jax.experimental import pallas as pl
from jax.experimental.pallas import tpu as pltpu
from jax.experimental.pallas import tpu_sc as plsc

_B, _C, _H, _W = 8, 19, 512, 512
_P = _H * _W
_N = _B * _P
_L = _B * _C * _P
_HB = 32
_NHB = _H // _HB
_SMOOTH = 1e-06

_NW = 32
_RW = _N // _NW
_ROUND = 1024
_NR = _RW // _ROUND
_NBINS = _C * 16


def _tc_body(x_ref, probs_ref, g_ref):
    b = pl.program_id(1)
    x = x_ref[0]
    m = jnp.max(x, axis=0)
    e = jnp.exp(x - m[None])
    s = jnp.sum(e, axis=0)
    p = e * (1.0 / s)[None]
    probs_ref[...] = p.reshape(_C, 128, 128)

    row_a = lax.broadcasted_iota(jnp.int32, (_C, _C * _HB), 0)
    col = lax.broadcasted_iota(jnp.int32, (_C, _C * _HB), 1)
    m0 = (((col // _HB) - (col % _HB)) % _C == row_a).astype(jnp.float32)
    p2 = p.reshape(_C * _HB, _W)
    a0 = lax.dot_general(m0, p2, (((1,), (0,)), ((), ())),
                         preferred_element_type=jnp.float32)
    wrow = lax.broadcasted_iota(jnp.int32, (_W, _C), 0)
    wcol = lax.broadcasted_iota(jnp.int32, (_W, _C), 1)
    w2 = (wrow % _C == wcol).astype(jnp.float32)
    g = lax.dot_general(a0, w2, (((1,), (0,)), ((), ())),
                        preferred_element_type=jnp.float32)

    @pl.when(b == 0)
    def _():
        g_ref[0] = jnp.zeros((_C, _C), jnp.float32)

    g_ref[0] += g


def _make_tc_call(b0, nb):
    return pl.pallas_call(
        _tc_body,
        grid=(_NHB, nb),
        in_specs=[pl.BlockSpec((1, _C, _HB, _W),
                               lambda hb, b: (b + b0, 0, hb, 0))],
        out_specs=[pl.BlockSpec((_C, 128, 128), lambda hb, b: (b, hb, 0)),
                   pl.BlockSpec((1, _C, _C), lambda hb, b: (hb, 0, 0))],
        out_shape=[jax.ShapeDtypeStruct((nb * _C, _H * _W // 128, 128),
                                        jnp.float32),
                   jax.ShapeDtypeStruct((_NHB, _C, _C), jnp.float32)],
        compiler_params=pltpu.CompilerParams(
            dimension_semantics=("arbitrary", "arbitrary")),
    )

_SEL = np.zeros((_NHB, _C, _C, _C), np.float32)
for _hb in range(_NHB):
    for _a in range(_C):
        for _d in range(_C):
            _SEL[_hb, _a, _d, (_a + _d - 32 * _hb) % _C] = 1.0


def _make_sc_body(rw):
  nr = rw // _ROUND

  def _sc_body(probs_hbm, t_hbm, iacc_hbm, cacc_hbm,
               t_v, idx_v, sidx_v, val_v, acc_v, cnt_v,
               tsem0, tsem1, gsem0, gsem1):
    wid = lax.axis_index("s") * 2 + lax.axis_index("c")
    base = wid * rw
    tsems = (tsem0, tsem1)
    gsems = (gsem0, gsem1)

    for u in range(_NBINS // 16):
        acc_v[pl.ds(u * 16, 16)] = jnp.zeros((16,), jnp.float32)
        cnt_v[pl.ds(u * 16, 16)] = jnp.zeros((16,), jnp.float32)

    lane = lax.iota(jnp.int32, 16)
    ones = jnp.ones((16,), jnp.float32)

    def tstart(r, slot):
        pltpu.async_copy(t_hbm.at[pl.ds(base + r * _ROUND, _ROUND)],
                         t_v.at[slot], tsems[slot])

    def twait(r, slot):
        pltpu.make_async_copy(t_hbm.at[pl.ds(base + r * _ROUND, _ROUND)],
                              t_v.at[slot], tsems[slot]).wait()

    def compidx(r, slot):
        start = base + r * _ROUND
        for v in range(_ROUND // 16):
            tv = t_v[slot, pl.ds(v * 16, 16)]
            q = start + v * 16 + lane
            idx_v[slot, pl.ds(v * 16, 16)] = q * 19 + tv
            sidx_v[slot, pl.ds(v * 16, 16)] = tv * 16 + lane

    def fire(slot):
        for j in range(_ROUND // 128):
            pltpu.async_copy(
                probs_hbm.at[idx_v.at[slot, pl.ds(j * 128, 128)]],
                val_v.at[slot, pl.ds(j * 128, 128)], gsems[slot])

    def drain(slot):
        for j in range(_ROUND // 128):
            pltpu.make_async_copy(
                probs_hbm.at[idx_v.at[slot, pl.ds(j * 128, 128)]],
                val_v.at[slot, pl.ds(j * 128, 128)], gsems[slot]).wait()

    def process(slot):
        for v in range(_ROUND // 16):
            sidx = sidx_v[slot, pl.ds(v * 16, 16)]
            val = val_v[slot, pl.ds(v * 16, 16)]
            plsc.addupdate_scatter(acc_v, [sidx], val)
            plsc.addupdate_scatter(cnt_v, [sidx], ones)

    def stage_a(r, slot, prefetch_next):
        twait(r, slot)
        compidx(r, slot)
        fire(slot)
        if prefetch_next:
            tstart(r + 1, 1 - slot)

    def stage_b(slot):
        drain(slot)
        process(slot)

    tstart(0, 0)
    stage_a(0, 0, True)

    def pair(pr, carry):
        r1 = 2 * pr + 1
        stage_a(r1, 1, True)
        stage_b(0)
        stage_a(r1 + 1, 0, True)
        stage_b(1)
        return carry

    lax.fori_loop(0, nr // 2 - 1, pair, 0)

    stage_a(nr - 1, 1, False)
    stage_b(0)
    stage_b(1)

    pltpu.sync_copy(acc_v, iacc_hbm.at[wid])
    pltpu.sync_copy(cnt_v, cacc_hbm.at[wid])

  return _sc_body


@functools.lru_cache(maxsize=None)
def _sc_call_cached(rw):
    return pl.kernel(
        _make_sc_body(rw),
        out_type=[jax.ShapeDtypeStruct((_NW, _NBINS), jnp.float32),
                  jax.ShapeDtypeStruct((_NW, _NBINS), jnp.float32)],
        mesh=plsc.VectorSubcoreMesh(core_axis_name="c", subcore_axis_name="s",
                                    num_cores=2, num_subcores=16),
        compiler_params=pltpu.CompilerParams(needs_layout_passes=False),
        scratch_types=[
            pltpu.VMEM((2, _ROUND), jnp.int32),
            pltpu.VMEM((2, _ROUND), jnp.int32),
            pltpu.VMEM((2, _ROUND), jnp.int32),
            pltpu.VMEM((2, _ROUND), jnp.float32),
            pltpu.VMEM((_NBINS,), jnp.float32),
            pltpu.VMEM((_NBINS,), jnp.float32),
            pltpu.SemaphoreType.DMA,
            pltpu.SemaphoreType.DMA,
            pltpu.SemaphoreType.DMA,
            pltpu.SemaphoreType.DMA,
        ],
    )


_NSPLIT = 2
_BS = _B // _NSPLIT


def kernel(logits, targets):
    t1d = jnp.reshape(targets, (_N,))
    rows = _N // _NSPLIT
    inter = jnp.zeros((_C,), jnp.float32)
    cnt = jnp.zeros((_C,), jnp.float32)
    gsum = jnp.zeros((_NHB, _C, _C), jnp.float32)
    for i in range(_NSPLIT):
        probs3, gacc = _make_tc_call(i * _BS, _BS)(logits)
        probs1d = jnp.reshape(probs3, (rows * _C,))
        iacc, cacc = _sc_call_cached(rows // _NW)(
            probs1d, lax.slice(t1d, (i * rows,), ((i + 1) * rows,)))
        inter = inter + iacc.reshape(_NW, _C, 16).sum(axis=(0, 2))
        cnt = cnt + cacc.reshape(_NW, _C, 16).sum(axis=(0, 2))
        gsum = gsum + gacc
    colsum = jnp.einsum("had,hadk->k", gsum, jnp.asarray(_SEL))
    union = colsum + cnt
    dice = 1.0 - (2.0 * inter + _SMOOTH) / (union + _SMOOTH)
    return dice.mean()


# speedup vs baseline: 69.1204x; 1.1633x over previous
import functools

import numpy as np
import jax
import jax.numpy as jnp
from jax import lax
from jax.experimental import pallas as pl
from jax.experimental.pallas import tpu as pltpu
from jax.experimental.pallas import tpu_sc as plsc

_B, _C, _H, _W = 8, 19, 512, 512
_P = _H * _W
_N = _B * _P
_L = _B * _C * _P
_HB = 64
_NHB = _H // _HB
_SMOOTH = 1e-06

_NW = 32
_RW = _N // _NW
_ROUND = 1024
_NR = _RW // _ROUND
_NBINS = _C * 16


def _tc_body(x_ref, probs_ref, g_ref):
    b = pl.program_id(1)
    x = x_ref[0]
    m = jnp.max(x, axis=0)
    e = jnp.exp(x - m[None])
    s = jnp.sum(e, axis=0)
    p = e * (1.0 / s)[None]
    probs_ref[...] = p.reshape(_C, _HB * 4, 128)

    row_a = lax.broadcasted_iota(jnp.int32, (_C, _C * _HB), 0)
    col = lax.broadcasted_iota(jnp.int32, (_C, _C * _HB), 1)
    m0 = (((col // _HB) - (col % _HB)) % _C == row_a).astype(jnp.float32)
    p2 = p.reshape(_C * _HB, _W)
    a0 = lax.dot_general(m0, p2, (((1,), (0,)), ((), ())),
                         preferred_element_type=jnp.float32)
    wrow = lax.broadcasted_iota(jnp.int32, (_W, _C), 0)
    wcol = lax.broadcasted_iota(jnp.int32, (_W, _C), 1)
    w2 = (wrow % _C == wcol).astype(jnp.float32)
    g = lax.dot_general(a0, w2, (((1,), (0,)), ((), ())),
                        preferred_element_type=jnp.float32)

    @pl.when(b == 0)
    def _():
        g_ref[0] = jnp.zeros((_C, _C), jnp.float32)

    g_ref[0] += g


def _make_tc_call(b0, nb):
    return pl.pallas_call(
        _tc_body,
        grid=(_NHB, nb),
        in_specs=[pl.BlockSpec((1, _C, _HB, _W),
                               lambda hb, b: (b + b0, 0, hb, 0))],
        out_specs=[pl.BlockSpec((_C, _HB * 4, 128), lambda hb, b: (b, hb, 0)),
                   pl.BlockSpec((1, _C, _C), lambda hb, b: (hb, 0, 0))],
        out_shape=[jax.ShapeDtypeStruct((nb * _C, _H * _W // 128, 128),
                                        jnp.float32),
                   jax.ShapeDtypeStruct((_NHB, _C, _C), jnp.float32)],
        compiler_params=pltpu.CompilerParams(
            dimension_semantics=("arbitrary", "arbitrary")),
    )

_SEL = np.zeros((_NHB, _C, _C, _C), np.float32)
for _hb in range(_NHB):
    for _a in range(_C):
        for _d in range(_C):
            _SEL[_hb, _a, _d, (_a + _d - _HB * _hb) % _C] = 1.0


def _make_sc_body(rw):
  nr = rw // _ROUND

  def _sc_body(probs_hbm, t_hbm, iacc_hbm, cacc_hbm,
               t_v, idx_v, sidx_v, val_v, acc_v, cnt_v,
               tsem0, tsem1, gsem0, gsem1):
    wid = lax.axis_index("s") * 2 + lax.axis_index("c")
    base = wid * rw
    tsems = (tsem0, tsem1)
    gsems = (gsem0, gsem1)

    for u in range(_NBINS // 16):
        acc_v[pl.ds(u * 16, 16)] = jnp.zeros((16,), jnp.float32)
        cnt_v[pl.ds(u * 16, 16)] = jnp.zeros((16,), jnp.float32)

    lane = lax.iota(jnp.int32, 16)
    ones = jnp.ones((16,), jnp.float32)

    def tstart(r, slot):
        pltpu.async_copy(t_hbm.at[pl.ds(base + r * _ROUND, _ROUND)],
                         t_v.at[slot], tsems[slot])

    def twait(r, slot):
        pltpu.make_async_copy(t_hbm.at[pl.ds(base + r * _ROUND, _ROUND)],
                              t_v.at[slot], tsems[slot]).wait()

    def compidx(r, slot):
        start = base + r * _ROUND
        for v in range(_ROUND // 16):
            tv = t_v[slot, pl.ds(v * 16, 16)]
            q = start + v * 16 + lane
            idx_v[slot, pl.ds(v * 16, 16)] = q * 19 + tv
            sidx_v[slot, pl.ds(v * 16, 16)] = tv * 16 + lane

    def fire(slot):
        for j in range(_ROUND // 128):
            pltpu.async_copy(
                probs_hbm.at[idx_v.at[slot, pl.ds(j * 128, 128)]],
                val_v.at[slot, pl.ds(j * 128, 128)], gsems[slot])

    def drain(slot):
        for j in range(_ROUND // 128):
            pltpu.make_async_copy(
                probs_hbm.at[idx_v.at[slot, pl.ds(j * 128, 128)]],
                val_v.at[slot, pl.ds(j * 128, 128)], gsems[slot]).wait()

    def process(slot):
        for v in range(_ROUND // 16):
            sidx = sidx_v[slot, pl.ds(v * 16, 16)]
            val = val_v[slot, pl.ds(v * 16, 16)]
            plsc.addupdate_scatter(acc_v, [sidx], val)
            plsc.addupdate_scatter(cnt_v, [sidx], ones)

    def stage_a(r, slot, prefetch_next):
        twait(r, slot)
        compidx(r, slot)
        fire(slot)
        if prefetch_next:
            tstart(r + 1, 1 - slot)

    def stage_b(slot):
        drain(slot)
        process(slot)

    tstart(0, 0)
    stage_a(0, 0, True)

    def pair(pr, carry):
        r1 = 2 * pr + 1
        stage_a(r1, 1, True)
        stage_b(0)
        stage_a(r1 + 1, 0, True)
        stage_b(1)
        return carry

    lax.fori_loop(0, nr // 2 - 1, pair, 0)

    stage_a(nr - 1, 1, False)
    stage_b(0)
    stage_b(1)

    pltpu.sync_copy(acc_v, iacc_hbm.at[wid])
    pltpu.sync_copy(cnt_v, cacc_hbm.at[wid])

  return _sc_body


@functools.lru_cache(maxsize=None)
def _sc_call_cached(rw):
    return pl.kernel(
        _make_sc_body(rw),
        out_type=[jax.ShapeDtypeStruct((_NW, _NBINS), jnp.float32),
                  jax.ShapeDtypeStruct((_NW, _NBINS), jnp.float32)],
        mesh=plsc.VectorSubcoreMesh(core_axis_name="c", subcore_axis_name="s",
                                    num_cores=2, num_subcores=16),
        compiler_params=pltpu.CompilerParams(needs_layout_passes=False),
        scratch_types=[
            pltpu.VMEM((2, _ROUND), jnp.int32),
            pltpu.VMEM((2, _ROUND), jnp.int32),
            pltpu.VMEM((2, _ROUND), jnp.int32),
            pltpu.VMEM((2, _ROUND), jnp.float32),
            pltpu.VMEM((_NBINS,), jnp.float32),
            pltpu.VMEM((_NBINS,), jnp.float32),
            pltpu.SemaphoreType.DMA,
            pltpu.SemaphoreType.DMA,
            pltpu.SemaphoreType.DMA,
            pltpu.SemaphoreType.DMA,
        ],
    )


_NSPLIT = 2
_BS = _B // _NSPLIT


def kernel(logits, targets):
    t1d = jnp.reshape(targets, (_N,))
    rows = _N // _NSPLIT
    inter = jnp.zeros((_C,), jnp.float32)
    cnt = jnp.zeros((_C,), jnp.float32)
    gsum = jnp.zeros((_NHB, _C, _C), jnp.float32)
    for i in range(_NSPLIT):
        probs3, gacc = _make_tc_call(i * _BS, _BS)(logits)
        probs1d = jnp.reshape(probs3, (rows * _C,))
        iacc, cacc = _sc_call_cached(rows // _NW)(
            probs1d, lax.slice(t1d, (i * rows,), ((i + 1) * rows,)))
        inter = inter + iacc.reshape(_NW, _C, 16).sum(axis=(0, 2))
        cnt = cnt + cacc.reshape(_NW, _C, 16).sum(axis=(0, 2))
        gsum = gsum + gacc
    colsum = jnp.einsum("had,hadk->k", gsum, jnp.asarray(_SEL))
    union = colsum + cnt
    dice = 1.0 - (2.0 * inter + _SMOOTH) / (union + _SMOOTH)
    return dice.mean()


# speedup vs baseline: 70.6412x; 1.0220x over previous
import functools

import numpy as np
import jax
import jax.numpy as jnp
from jax import lax
from jax.experimental import pallas as pl
from jax.experimental.pallas import tpu as pltpu
from jax.experimental.pallas import tpu_sc as plsc

_B, _C, _H, _W = 8, 19, 512, 512
_P = _H * _W
_N = _B * _P
_L = _B * _C * _P
_HB = 128
_NHB = _H // _HB
_SMOOTH = 1e-06

_NW = 32
_RW = _N // _NW
_ROUND = 1024
_NR = _RW // _ROUND
_NBINS = _C * 16


def _tc_body(x_ref, probs_ref, g_ref):
    b = pl.program_id(1)
    x = x_ref[0]
    m = jnp.max(x, axis=0)
    e = jnp.exp(x - m[None])
    s = jnp.sum(e, axis=0)
    p = e * (1.0 / s)[None]
    probs_ref[...] = p.reshape(_C, _HB * 4, 128)

    row_a = lax.broadcasted_iota(jnp.int32, (_C, _C * _HB), 0)
    col = lax.broadcasted_iota(jnp.int32, (_C, _C * _HB), 1)
    m0 = (((col // _HB) - (col % _HB)) % _C == row_a).astype(jnp.float32)
    p2 = p.reshape(_C * _HB, _W)
    a0 = lax.dot_general(m0, p2, (((1,), (0,)), ((), ())),
                         preferred_element_type=jnp.float32)
    wrow = lax.broadcasted_iota(jnp.int32, (_W, _C), 0)
    wcol = lax.broadcasted_iota(jnp.int32, (_W, _C), 1)
    w2 = (wrow % _C == wcol).astype(jnp.float32)
    g = lax.dot_general(a0, w2, (((1,), (0,)), ((), ())),
                        preferred_element_type=jnp.float32)

    @pl.when(b == 0)
    def _():
        g_ref[0] = jnp.zeros((_C, _C), jnp.float32)

    g_ref[0] += g


def _make_tc_call(b0, nb):
    return pl.pallas_call(
        _tc_body,
        grid=(_NHB, nb),
        in_specs=[pl.BlockSpec((1, _C, _HB, _W),
                               lambda hb, b: (b + b0, 0, hb, 0))],
        out_specs=[pl.BlockSpec((_C, _HB * 4, 128), lambda hb, b: (b, hb, 0)),
                   pl.BlockSpec((1, _C, _C), lambda hb, b: (hb, 0, 0))],
        out_shape=[jax.ShapeDtypeStruct((nb * _C, _H * _W // 128, 128),
                                        jnp.float32),
                   jax.ShapeDtypeStruct((_NHB, _C, _C), jnp.float32)],
        compiler_params=pltpu.CompilerParams(
            dimension_semantics=("arbitrary", "arbitrary")),
    )

_SEL = np.zeros((_NHB, _C, _C, _C), np.float32)
for _hb in range(_NHB):
    for _a in range(_C):
        for _d in range(_C):
            _SEL[_hb, _a, _d, (_a + _d - _HB * _hb) % _C] = 1.0


def _make_sc_body(rw):
  nr = rw // _ROUND

  def _sc_body(probs_hbm, t_hbm, iacc_hbm, cacc_hbm,
               t_v, idx_v, sidx_v, val_v, acc_v, cnt_v,
               tsem0, tsem1, gsem0, gsem1):
    wid = lax.axis_index("s") * 2 + lax.axis_index("c")
    base = wid * rw
    tsems = (tsem0, tsem1)
    gsems = (gsem0, gsem1)

    for u in range(_NBINS // 16):
        acc_v[pl.ds(u * 16, 16)] = jnp.zeros((16,), jnp.float32)
        cnt_v[pl.ds(u * 16, 16)] = jnp.zeros((16,), jnp.float32)

    lane = lax.iota(jnp.int32, 16)
    ones = jnp.ones((16,), jnp.float32)

    def tstart(r, slot):
        pltpu.async_copy(t_hbm.at[pl.ds(base + r * _ROUND, _ROUND)],
                         t_v.at[slot], tsems[slot])

    def twait(r, slot):
        pltpu.make_async_copy(t_hbm.at[pl.ds(base + r * _ROUND, _ROUND)],
                              t_v.at[slot], tsems[slot]).wait()

    def compidx(r, slot):
        start = base + r * _ROUND
        for v in range(_ROUND // 16):
            tv = t_v[slot, pl.ds(v * 16, 16)]
            q = start + v * 16 + lane
            idx_v[slot, pl.ds(v * 16, 16)] = q * 19 + tv
            sidx_v[slot, pl.ds(v * 16, 16)] = tv * 16 + lane

    def fire(slot):
        for j in range(_ROUND // 128):
            pltpu.async_copy(
                probs_hbm.at[idx_v.at[slot, pl.ds(j * 128, 128)]],
                val_v.at[slot, pl.ds(j * 128, 128)], gsems[slot])

    def drain(slot):
        for j in range(_ROUND // 128):
            pltpu.make_async_copy(
                probs_hbm.at[idx_v.at[slot, pl.ds(j * 128, 128)]],
                val_v.at[slot, pl.ds(j * 128, 128)], gsems[slot]).wait()

    def process(slot):
        for v in range(_ROUND // 16):
            sidx = sidx_v[slot, pl.ds(v * 16, 16)]
            val = val_v[slot, pl.ds(v * 16, 16)]
            plsc.addupdate_scatter(acc_v, [sidx], val)
            plsc.addupdate_scatter(cnt_v, [sidx], ones)

    def stage_a(r, slot, prefetch_next):
        twait(r, slot)
        compidx(r, slot)
        fire(slot)
        if prefetch_next:
            tstart(r + 1, 1 - slot)

    def stage_b(slot):
        drain(slot)
        process(slot)

    tstart(0, 0)
    stage_a(0, 0, True)

    def pair(pr, carry):
        r1 = 2 * pr + 1
        stage_a(r1, 1, True)
        stage_b(0)
        stage_a(r1 + 1, 0, True)
        stage_b(1)
        return carry

    lax.fori_loop(0, nr // 2 - 1, pair, 0)

    stage_a(nr - 1, 1, False)
    stage_b(0)
    stage_b(1)

    pltpu.sync_copy(acc_v, iacc_hbm.at[wid])
    pltpu.sync_copy(cnt_v, cacc_hbm.at[wid])

  return _sc_body


@functools.lru_cache(maxsize=None)
def _sc_call_cached(rw):
    return pl.kernel(
        _make_sc_body(rw),
        out_type=[jax.ShapeDtypeStruct((_NW, _NBINS), jnp.float32),
                  jax.ShapeDtypeStruct((_NW, _NBINS), jnp.float32)],
        mesh=plsc.VectorSubcoreMesh(core_axis_name="c", subcore_axis_name="s",
                                    num_cores=2, num_subcores=16),
        compiler_params=pltpu.CompilerParams(needs_layout_passes=False),
        scratch_types=[
            pltpu.VMEM((2, _ROUND), jnp.int32),
            pltpu.VMEM((2, _ROUND), jnp.int32),
            pltpu.VMEM((2, _ROUND), jnp.int32),
            pltpu.VMEM((2, _ROUND), jnp.float32),
            pltpu.VMEM((_NBINS,), jnp.float32),
            pltpu.VMEM((_NBINS,), jnp.float32),
            pltpu.SemaphoreType.DMA,
            pltpu.SemaphoreType.DMA,
            pltpu.SemaphoreType.DMA,
            pltpu.SemaphoreType.DMA,
        ],
    )


_NSPLIT = 2
_BS = _B // _NSPLIT


def kernel(logits, targets):
    t1d = jnp.reshape(targets, (_N,))
    rows = _N // _NSPLIT
    inter = jnp.zeros((_C,), jnp.float32)
    cnt = jnp.zeros((_C,), jnp.float32)
    gsum = jnp.zeros((_NHB, _C, _C), jnp.float32)
    for i in range(_NSPLIT):
        probs3, gacc = _make_tc_call(i * _BS, _BS)(logits)
        probs1d = jnp.reshape(probs3, (rows * _C,))
        iacc, cacc = _sc_call_cached(rows // _NW)(
            probs1d, lax.slice(t1d, (i * rows,), ((i + 1) * rows,)))
        inter = inter + iacc.reshape(_NW, _C, 16).sum(axis=(0, 2))
        cnt = cnt + cacc.reshape(_NW, _C, 16).sum(axis=(0, 2))
        gsum = gsum + gacc
    colsum = jnp.einsum("had,hadk->k", gsum, jnp.asarray(_SEL))
    union = colsum + cnt
    dice = 1.0 - (2.0 * inter + _SMOOTH) / (union + _SMOOTH)
    return dice.mean()
